# Initial kernel scaffold; baseline (speedup 1.0000x reference)
import jax
import jax.numpy as jnp
from jax.experimental import pallas as pl


def kernel(x, edge_index, weight, w_comp, h_bias):
    raise NotImplementedError("write your pallas kernel here")


# speedup vs baseline: 3.4094x; 3.4094x over previous
import functools
import jax
import jax.numpy as jnp
from jax import lax
from jax.experimental import pallas as pl
from jax.experimental.pallas import tpu as pltpu
from jax.experimental.pallas import tpu_sc as plsc

N = 10000
D = 128
R = 4
B = 2
E = 80000

NSC = 2
NTILE = 16
RELS_PER_SC = R // NSC
EP_TILE = E // NTILE
CHUNK = 128
NCHUNK = EP_TILE // CHUNK + 1
EP_PAD = NCHUNK * CHUNK
NPAD = 10240
STRIPE = NPAD // NTILE
CW = 16
H = D // 2


def _sc_body(x2_hbm, src_hbm, dst_hbm, zrow_hbm, zcnt_hbm, ones_hbm,
             sums_hbm, cnt_hbm,
             src_v, dst_v, buf_a, buf_b, ones_v,
             accum_sh, cnt_sh, sem_a, sem_b):
  c = lax.axis_index("c")
  s = lax.axis_index("s")
  pltpu.sync_copy(ones_hbm, ones_v)
  for k in range(RELS_PER_SC):
    rel = c * RELS_PER_SC + k
    pltpu.sync_copy(dst_hbm.at[rel, s], dst_v)
    for p in range(2):
      pltpu.sync_copy(zrow_hbm, accum_sh.at[pl.ds(s * STRIPE, STRIPE)])
      if p == 0:
        pltpu.sync_copy(zcnt_hbm, cnt_sh.at[pl.ds(s * STRIPE, STRIPE)])
      pltpu.sync_copy(src_hbm.at[p, rel, s], src_v)
      plsc.subcore_barrier()

      pltpu.async_copy(x2_hbm.at[src_v.at[0]], buf_a, sem_a)

      def body(i, carry):
        ca = 2 * i
        cb = 2 * i + 1
        pltpu.async_copy(x2_hbm.at[src_v.at[cb]], buf_b, sem_b)
        pltpu.make_async_copy(x2_hbm.at[src_v.at[ca]], buf_a, sem_a).wait()
        pltpu.sync_copy(buf_a, accum_sh.at[dst_v.at[ca]], add=True)
        if p == 0:
          pltpu.sync_copy(ones_v, cnt_sh.at[dst_v.at[ca]], add=True)

        @pl.when(i < NCHUNK // 2 - 1)
        def _():
          pltpu.async_copy(x2_hbm.at[src_v.at[ca + 2]], buf_a, sem_a)

        pltpu.make_async_copy(x2_hbm.at[src_v.at[cb]], buf_b, sem_b).wait()
        pltpu.sync_copy(buf_b, accum_sh.at[dst_v.at[cb]], add=True)
        if p == 0:
          pltpu.sync_copy(ones_v, cnt_sh.at[dst_v.at[cb]], add=True)
        return carry

      lax.fori_loop(0, NCHUNK // 2, body, 0)
      plsc.subcore_barrier()
      pltpu.sync_copy(accum_sh.at[pl.ds(s * STRIPE, STRIPE)],
                      sums_hbm.at[p, rel, pl.ds(s * STRIPE, STRIPE)])
      if p == 0:
        pltpu.sync_copy(cnt_sh.at[pl.ds(s * STRIPE, STRIPE)],
                        cnt_hbm.at[rel, pl.ds(s * STRIPE, STRIPE)])


@functools.cache
def _sc_aggregate_fn():
  return pl.kernel(
      _sc_body,
      out_type=(
          jax.ShapeDtypeStruct((2, R, NPAD, H), jnp.float32),
          jax.ShapeDtypeStruct((R, NPAD, CW), jnp.float32),
      ),
      mesh=plsc.VectorSubcoreMesh(core_axis_name="c", subcore_axis_name="s",
                                  num_cores=NSC, num_subcores=NTILE),
      compiler_params=pltpu.CompilerParams(use_tc_tiling_on_sc=False),
      scratch_types=[
          pltpu.VMEM((NCHUNK, CHUNK), jnp.int32),
          pltpu.VMEM((NCHUNK, CHUNK), jnp.int32),
          pltpu.VMEM((CHUNK, H), jnp.float32),
          pltpu.VMEM((CHUNK, H), jnp.float32),
          pltpu.VMEM((CHUNK, CW), jnp.float32),
          pltpu.VMEM_SHARED((NPAD, H), jnp.float32),
          pltpu.VMEM_SHARED((NPAD, CW), jnp.float32),
          pltpu.SemaphoreType.DMA,
          pltpu.SemaphoreType.DMA,
      ],
  )


BLK = 1280


def _tc_body(sums_ref, cnt_ref, w_ref, wcomp_ref, bias_ref, out_ref):
  acc = jnp.zeros((BLK, D), jnp.float32)
  w0 = w_ref[0]
  w1 = w_ref[1]
  for r in range(R):
    wr = wcomp_ref[r, 0] * w0 + wcomp_ref[r, 1] * w1
    cnt = jnp.maximum(cnt_ref[r][:, 0:1], 1.0)
    mean = jnp.concatenate([sums_ref[0, r], sums_ref[1, r]], axis=1) / cnt
    acc = acc + jnp.dot(mean, wr, preferred_element_type=jnp.float32)
  out_ref[...] = acc + bias_ref[0]


def _tc_combine(sums, cnt, weight, w_comp, h_bias):
  return pl.pallas_call(
      _tc_body,
      grid=(NPAD // BLK,),
      in_specs=[
          pl.BlockSpec((2, R, BLK, H), lambda i: (0, 0, i, 0)),
          pl.BlockSpec((R, BLK, CW), lambda i: (0, i, 0)),
          pl.BlockSpec((B, D, D), lambda i: (0, 0, 0)),
          pl.BlockSpec(memory_space=pltpu.SMEM),
          pl.BlockSpec((1, D), lambda i: (0, 0)),
      ],
      out_specs=pl.BlockSpec((BLK, D), lambda i: (i, 0)),
      out_shape=jax.ShapeDtypeStruct((NPAD, D), jnp.float32),
  )(sums, cnt, weight, w_comp, h_bias.reshape(1, D))


@jax.jit
def kernel(x, edge_index, weight, w_comp, h_bias):
  src = edge_index[:, 0, :].reshape(R, NTILE, EP_TILE)
  dst = edge_index[:, 1, :].reshape(R, NTILE, EP_TILE)
  pad = EP_PAD - EP_TILE
  src = jnp.pad(src, ((0, 0), (0, 0), (0, pad))).reshape(R, NTILE, NCHUNK, CHUNK)
  src2 = jnp.stack([2 * src, 2 * src + 1])
  dst = jnp.pad(dst, ((0, 0), (0, 0), (0, pad)), constant_values=N)
  dst = dst.reshape(R, NTILE, NCHUNK, CHUNK)
  x2 = x.reshape(2 * N, H)
  zrow = jnp.zeros((STRIPE, H), jnp.float32)
  zcnt = jnp.zeros((STRIPE, CW), jnp.float32)
  ones = jnp.ones((CHUNK, CW), jnp.float32)

  sums, cnt = _sc_aggregate_fn()(x2, src2, dst, zrow, zcnt, ones)
  h = _tc_combine(sums, cnt, weight, w_comp, h_bias)
  return h[:N]


# speedup vs baseline: 3.5165x; 1.0314x over previous
import functools
import jax
import jax.numpy as jnp
from jax import lax
from jax.experimental import pallas as pl
from jax.experimental.pallas import tpu as pltpu
from jax.experimental.pallas import tpu_sc as plsc

N = 10000
D = 128
R = 4
B = 2
E = 80000

NSC = 2
NTILE = 16
RELS_PER_SC = R // NSC
EP_TILE = E // NTILE
CHUNK = 128
NCHUNK = EP_TILE // CHUNK + 1
EP_PAD = NCHUNK * CHUNK
NPAD = 10240
STRIPE = NPAD // NTILE
CW = 16
H = D // 2


NBUF = 4
NGROUP = NCHUNK // NBUF


def _sc_body(x2_hbm, src_hbm, dst_hbm, zrow_hbm, zcnt_hbm, ones_hbm,
             sums_hbm, cnt_hbm,
             src_v, dst_v, bufs, ones_v,
             accum_sh, cnt_sh, gsem, ssem, csem):
  c = lax.axis_index("c")
  s = lax.axis_index("s")
  pltpu.sync_copy(ones_hbm, ones_v)
  for k in range(RELS_PER_SC):
    rel = c * RELS_PER_SC + k
    pltpu.sync_copy(dst_hbm.at[rel, s], dst_v)
    for p in range(2):
      pltpu.sync_copy(zrow_hbm, accum_sh.at[pl.ds(s * STRIPE, STRIPE)])
      if p == 0:
        pltpu.sync_copy(zcnt_hbm, cnt_sh.at[pl.ds(s * STRIPE, STRIPE)])
      pltpu.sync_copy(src_hbm.at[p, rel, s], src_v)
      plsc.subcore_barrier()

      def gather(chunk, b):
        return pltpu.async_copy(x2_hbm.at[src_v.at[chunk]], bufs.at[b], gsem.at[b])

      def scat(chunk, b):
        return pltpu.async_copy(bufs.at[b], accum_sh.at[dst_v.at[chunk]],
                                ssem.at[b], add=True)

      def cnt_scat(chunk, b):
        return pltpu.async_copy(ones_v, cnt_sh.at[dst_v.at[chunk]],
                                csem.at[b], add=True)

      for b in range(NBUF):
        gather(b, b)

      def body(g, carry):
        base = g * NBUF
        for b in range(NBUF):
          ch = base + b
          pltpu.make_async_copy(x2_hbm.at[src_v.at[ch]], bufs.at[b],
                                gsem.at[b]).wait()
          scat(ch, b)
          if p == 0:
            cnt_scat(ch, b)
        for b in range(NBUF):
          ch = base + b

          @pl.when(g < NGROUP - 1)
          def _():
            pltpu.make_async_copy(bufs.at[b], accum_sh.at[dst_v.at[ch]],
                                  ssem.at[b]).wait()
            if p == 0:
              pltpu.make_async_copy(ones_v, cnt_sh.at[dst_v.at[ch]],
                                    csem.at[b]).wait()
            gather(ch + NBUF, b)
        return carry

      lax.fori_loop(0, NGROUP, body, 0)
      last = (NGROUP - 1) * NBUF
      for b in range(NBUF):
        pltpu.make_async_copy(bufs.at[b], accum_sh.at[dst_v.at[last + b]],
                              ssem.at[b]).wait()
        if p == 0:
          pltpu.make_async_copy(ones_v, cnt_sh.at[dst_v.at[last + b]],
                                csem.at[b]).wait()
      plsc.subcore_barrier()
      pltpu.sync_copy(accum_sh.at[pl.ds(s * STRIPE, STRIPE)],
                      sums_hbm.at[p, rel, pl.ds(s * STRIPE, STRIPE)])
      if p == 0:
        pltpu.sync_copy(cnt_sh.at[pl.ds(s * STRIPE, STRIPE)],
                        cnt_hbm.at[rel, pl.ds(s * STRIPE, STRIPE)])


@functools.cache
def _sc_aggregate_fn():
  return pl.kernel(
      _sc_body,
      out_type=(
          jax.ShapeDtypeStruct((2, R, NPAD, H), jnp.float32),
          jax.ShapeDtypeStruct((R, NPAD, CW), jnp.float32),
      ),
      mesh=plsc.VectorSubcoreMesh(core_axis_name="c", subcore_axis_name="s",
                                  num_cores=NSC, num_subcores=NTILE),
      compiler_params=pltpu.CompilerParams(use_tc_tiling_on_sc=False),
      scratch_types=[
          pltpu.VMEM((NCHUNK, CHUNK), jnp.int32),
          pltpu.VMEM((NCHUNK, CHUNK), jnp.int32),
          pltpu.VMEM((NBUF, CHUNK, H), jnp.float32),
          pltpu.VMEM((CHUNK, CW), jnp.float32),
          pltpu.VMEM_SHARED((NPAD, H), jnp.float32),
          pltpu.VMEM_SHARED((NPAD, CW), jnp.float32),
          pltpu.SemaphoreType.DMA((NBUF,)),
          pltpu.SemaphoreType.DMA((NBUF,)),
          pltpu.SemaphoreType.DMA((NBUF,)),
      ],
  )


BLK = 1280


def _tc_body(sums_ref, cnt_ref, w_ref, wcomp_ref, bias_ref, out_ref):
  acc = jnp.zeros((BLK, D), jnp.float32)
  w0 = w_ref[0]
  w1 = w_ref[1]
  for r in range(R):
    wr = wcomp_ref[r, 0] * w0 + wcomp_ref[r, 1] * w1
    cnt = jnp.maximum(cnt_ref[r][:, 0:1], 1.0)
    mean = jnp.concatenate([sums_ref[0, r], sums_ref[1, r]], axis=1) / cnt
    acc = acc + jnp.dot(mean, wr, preferred_element_type=jnp.float32)
  out_ref[...] = acc + bias_ref[0]


def _tc_combine(sums, cnt, weight, w_comp, h_bias):
  return pl.pallas_call(
      _tc_body,
      grid=(NPAD // BLK,),
      in_specs=[
          pl.BlockSpec((2, R, BLK, H), lambda i: (0, 0, i, 0)),
          pl.BlockSpec((R, BLK, CW), lambda i: (0, i, 0)),
          pl.BlockSpec((B, D, D), lambda i: (0, 0, 0)),
          pl.BlockSpec(memory_space=pltpu.SMEM),
          pl.BlockSpec((1, D), lambda i: (0, 0)),
      ],
      out_specs=pl.BlockSpec((BLK, D), lambda i: (i, 0)),
      out_shape=jax.ShapeDtypeStruct((NPAD, D), jnp.float32),
  )(sums, cnt, weight, w_comp, h_bias.reshape(1, D))


@jax.jit
def kernel(x, edge_index, weight, w_comp, h_bias):
  src = edge_index[:, 0, :].reshape(R, NTILE, EP_TILE)
  dst = edge_index[:, 1, :].reshape(R, NTILE, EP_TILE)
  pad = EP_PAD - EP_TILE
  src = jnp.pad(src, ((0, 0), (0, 0), (0, pad))).reshape(R, NTILE, NCHUNK, CHUNK)
  src2 = jnp.stack([2 * src, 2 * src + 1])
  dst = jnp.pad(dst, ((0, 0), (0, 0), (0, pad)), constant_values=N)
  dst = dst.reshape(R, NTILE, NCHUNK, CHUNK)
  x2 = x.reshape(2 * N, H)
  zrow = jnp.zeros((STRIPE, H), jnp.float32)
  zcnt = jnp.zeros((STRIPE, CW), jnp.float32)
  ones = jnp.ones((CHUNK, CW), jnp.float32)

  sums, cnt = _sc_aggregate_fn()(x2, src2, dst, zrow, zcnt, ones)
  h = _tc_combine(sums, cnt, weight, w_comp, h_bias)
  return h[:N]
